# Initial kernel scaffold; baseline (speedup 1.0000x reference)
import jax
import jax.numpy as jnp
from jax.experimental import pallas as pl


def kernel(input):
    raise NotImplementedError("write your pallas kernel here")


# speedup vs baseline: 86.4945x; 86.4945x over previous
import functools

import jax
import jax.numpy as jnp
from jax import lax
from jax.experimental import pallas as pl
from jax.experimental.pallas import tpu as pltpu
from jax.experimental.pallas import tpu_sc as plsc

N, SF, C, H, W = 4, 8, 3, 512, 512
NUM_BINS = 256
M = N * SF * C
PIX = H * W
NC, NS, L = 2, 16, 16
NW = NC * NS
ROWS_PER_W = M // NW
CHUNK = 16384
NCHUNK = PIX // CHUNK
UNROLL = 8

_mesh = plsc.VectorSubcoreMesh(core_axis_name="c", subcore_axis_name="s")


@functools.partial(
    pl.kernel,
    out_type=jax.ShapeDtypeStruct((M * NUM_BINS,), jnp.float32),
    mesh=_mesh,
    scratch_types=[
        pltpu.VMEM((CHUNK,), jnp.float32),
        pltpu.VMEM((L * NUM_BINS,), jnp.float32),
        pltpu.VMEM((NUM_BINS,), jnp.float32),
    ],
    compiler_params=pltpu.CompilerParams(needs_layout_passes=False),
)
def _hist_kernel(x_hbm, out_hbm, buf, hist, outbuf):
    wid = lax.axis_index("s") * NC + lax.axis_index("c")
    lane_base = jnp.arange(L, dtype=jnp.int32) * NUM_BINS

    for r in range(ROWS_PER_W):
        row = wid * ROWS_PER_W + r

        def zero_body(i, _):
            hist[pl.ds(i * 16, 16)] = jnp.zeros((16,), jnp.float32)
            return 0

        lax.fori_loop(0, (L * NUM_BINS) // 16, zero_body, 0)

        base = row * PIX
        for g in range(NCHUNK):
            pltpu.sync_copy(x_hbm.at[pl.ds(base + g * CHUNK, CHUNK)], buf)

            def body(i, _):
                off = i * (UNROLL * 16)
                for u in range(UNROLL):
                    x = buf[pl.ds(off + u * 16, 16)]
                    xc = jnp.minimum(jnp.maximum(x, 0.0), 255.0)
                    lo_i = xc.astype(jnp.int32)
                    frac = xc - lo_i.astype(jnp.float32)
                    hi_i = jnp.minimum(lo_i + 1, NUM_BINS - 1)
                    plsc.addupdate_scatter(hist, [lane_base + lo_i], 1.0 - frac)
                    plsc.addupdate_scatter(hist, [lane_base + hi_i], frac)
                return 0

            lax.fori_loop(0, CHUNK // (UNROLL * 16), body, 0)

        def reduce_body(j, _):
            acc = jnp.zeros((16,), jnp.float32)
            for l in range(L):
                acc = acc + hist[pl.ds(l * NUM_BINS + j * 16, 16)]
            outbuf[pl.ds(j * 16, 16)] = acc
            return 0

        lax.fori_loop(0, NUM_BINS // 16, reduce_body, 0)
        pltpu.sync_copy(outbuf, out_hbm.at[pl.ds(row * NUM_BINS, NUM_BINS)])


def kernel(input):
    out = _hist_kernel(input.reshape(-1))
    return out.reshape(N, SF, C, NUM_BINS)


# speedup vs baseline: 251.0538x; 2.9025x over previous
import functools

import jax
import jax.numpy as jnp
from jax import lax
from jax.experimental import pallas as pl
from jax.experimental.pallas import tpu as pltpu
from jax.experimental.pallas import tpu_sc as plsc

N, SF, C, H, W = 4, 8, 3, 512, 512
NUM_BINS = 256
M = N * SF * C
PIX = H * W
NC, NS, L = 2, 16, 16
NW = NC * NS
ROWS_PER_W = M // NW
BSTRIDE = 272
CHUNK = 32768
NCHUNK = ROWS_PER_W * PIX // CHUNK
CHUNKS_PER_ROW = PIX // CHUNK
UNROLL = 8
HSIZE = ROWS_PER_W * L * BSTRIDE
OSIZE = ROWS_PER_W * NUM_BINS

_mesh = plsc.VectorSubcoreMesh(core_axis_name="c", subcore_axis_name="s")


@functools.partial(
    pl.kernel,
    out_type=jax.ShapeDtypeStruct((M * NUM_BINS,), jnp.float32),
    mesh=_mesh,
    scratch_types=[
        pltpu.VMEM((CHUNK,), jnp.float32),
        pltpu.VMEM((CHUNK,), jnp.float32),
        pltpu.VMEM((HSIZE,), jnp.float32),
        pltpu.VMEM((OSIZE,), jnp.float32),
        pltpu.SemaphoreType.DMA,
        pltpu.SemaphoreType.DMA,
    ],
    compiler_params=pltpu.CompilerParams(needs_layout_passes=False),
)
def _hist_kernel(x_hbm, out_hbm, buf0, buf1, hist, outbuf, sem0, sem1):
    wid = lax.axis_index("s") * NC + lax.axis_index("c")
    lane = jnp.arange(L, dtype=jnp.int32)

    def zero_body(i, _):
        hist[pl.ds(i * 16, 16)] = jnp.zeros((16,), jnp.float32)
        return 0

    lax.fori_loop(0, HSIZE // 16, zero_body, 0)

    base = wid * (ROWS_PER_W * PIX)
    bufs = (buf0, buf1)
    sems = (sem0, sem1)

    def start(g):
        return pltpu.async_copy(
            x_hbm.at[pl.ds(base + g * CHUNK, CHUNK)], bufs[g % 2], sems[g % 2]
        )

    pending = start(0)
    for g in range(NCHUNK):
        buf = bufs[g % 2]
        cur = pending
        if g + 1 < NCHUNK:
            pending = start(g + 1)
        cur.wait()

        r = g // CHUNKS_PER_ROW
        lane_base = (r * L + lane) * BSTRIDE

        @plsc.parallel_loop(0, CHUNK, step=UNROLL * 16)
        def _chunk_body(i):
            for u in range(UNROLL):
                x = buf[pl.ds(i + u * 16, 16)]
                lo_i = x.astype(jnp.int32)
                frac = x - lo_i.astype(jnp.float32)
                idx = lane_base + lo_i
                plsc.addupdate_scatter(hist, [idx], 1.0 - frac)
                plsc.addupdate_scatter(hist, [idx + 1], frac)

    for r in range(ROWS_PER_W):
        def reduce_body(j, _, r=r):
            acc = hist[pl.ds((r * L) * BSTRIDE + j * 16, 16)]
            for l in range(1, L):
                acc = acc + hist[pl.ds((r * L + l) * BSTRIDE + j * 16, 16)]
            outbuf[pl.ds(r * NUM_BINS + j * 16, 16)] = acc
            return 0

        lax.fori_loop(0, NUM_BINS // 16, reduce_body, 0)

    pltpu.sync_copy(outbuf, out_hbm.at[pl.ds(wid * OSIZE, OSIZE)])


def kernel(input):
    out = _hist_kernel(input.reshape(-1))
    return out.reshape(N, SF, C, NUM_BINS)


# speedup vs baseline: 307.9069x; 1.2265x over previous
import functools

import jax
import jax.numpy as jnp
from jax import lax
from jax.experimental import pallas as pl
from jax.experimental.pallas import tpu as pltpu
from jax.experimental.pallas import tpu_sc as plsc

N, SF, C, H, W = 4, 8, 3, 512, 512
NUM_BINS = 256
M = N * SF * C
PIX = H * W
NC, NS, L = 2, 16, 16
NW = NC * NS
ROWS_PER_W = M // NW
NBP = NUM_BINS + 1
CHUNK = 32768
NCHUNK = ROWS_PER_W * PIX // CHUNK
CHUNKS_PER_ROW = PIX // CHUNK
UNROLL = 8
HSIZE = ROWS_PER_W * NBP * L
OSIZE = ROWS_PER_W * NUM_BINS

_mesh = plsc.VectorSubcoreMesh(core_axis_name="c", subcore_axis_name="s")


@functools.partial(
    pl.kernel,
    out_type=jax.ShapeDtypeStruct((M * NUM_BINS,), jnp.float32),
    mesh=_mesh,
    scratch_types=[
        pltpu.VMEM((CHUNK,), jnp.float32),
        pltpu.VMEM((CHUNK,), jnp.float32),
        pltpu.VMEM((HSIZE,), jnp.float32),
        pltpu.VMEM((OSIZE,), jnp.float32),
        pltpu.SemaphoreType.DMA,
        pltpu.SemaphoreType.DMA,
    ],
    compiler_params=pltpu.CompilerParams(needs_layout_passes=False),
)
def _hist_kernel(x_hbm, out_hbm, buf0, buf1, hist, outbuf, sem0, sem1):
    wid = lax.axis_index("s") * NC + lax.axis_index("c")
    lane = jnp.arange(L, dtype=jnp.int32)

    def zero_body(i, _):
        hist[pl.ds(i * 16, 16)] = jnp.zeros((16,), jnp.float32)
        return 0

    lax.fori_loop(0, HSIZE // 16, zero_body, 0)

    base = wid * (ROWS_PER_W * PIX)
    bufs = (buf0, buf1)
    sems = (sem0, sem1)

    def start(g):
        return pltpu.async_copy(
            x_hbm.at[pl.ds(base + g * CHUNK, CHUNK)], bufs[g % 2], sems[g % 2]
        )

    pending = start(0)
    for g in range(NCHUNK):
        buf = bufs[g % 2]
        cur = pending
        if g + 1 < NCHUNK:
            pending = start(g + 1)
        cur.wait()

        r = g // CHUNKS_PER_ROW
        lane_base = lane + (r * NBP * L)

        @plsc.parallel_loop(0, CHUNK, step=UNROLL * 16)
        def _chunk_body(i):
            for u in range(UNROLL):
                x = buf[pl.ds(i + u * 16, 16)]
                lo_i = x.astype(jnp.int32)
                frac = x - lo_i.astype(jnp.float32)
                idx = lane_base + (lo_i << 4)
                plsc.addupdate_scatter(hist, [idx], 1.0 - frac)
                plsc.addupdate_scatter(hist, [idx + 16], frac)

    bin_word = lane * L
    for r in range(ROWS_PER_W):
        def reduce_body(j, _, r=r):
            base = (r * NBP + j * 16) * L + bin_word
            acc = jnp.zeros((16,), jnp.float32)
            for k in range(L):
                perm = (lane + k) & (L - 1)
                acc = acc + plsc.load_gather(hist, [base + perm])
            outbuf[pl.ds(r * NUM_BINS + j * 16, 16)] = acc
            return 0

        lax.fori_loop(0, NUM_BINS // 16, reduce_body, 0)

    pltpu.sync_copy(outbuf, out_hbm.at[pl.ds(wid * OSIZE, OSIZE)])


def kernel(input):
    out = _hist_kernel(input.reshape(-1))
    return out.reshape(N, SF, C, NUM_BINS)
